# baseline (speedup 1.0000x reference)
import functools

import jax
import jax.numpy as jnp
from jax import lax
from jax.experimental import pallas as pl
from jax.experimental.pallas import tpu as pltpu
from jax.experimental.pallas import tpu_sc as plsc

B, K, D = 2048, 256, 128
BLOCK_B = 64
HALVES = 4
BH = B // HALVES
GROUPS = BH // 16
NW = 32
GPW = GROUPS // (NW // 2)


def _scan_kernel(x_ref, c_ref, out1_ref, out2_ref):
    x = x_ref[...]

    inf = jnp.float32(jnp.inf)
    min1 = jnp.full((BLOCK_B, D), inf, jnp.float32)
    min2 = jnp.full((BLOCK_B, D), inf, jnp.float32)
    arg1 = jnp.zeros((BLOCK_B, D), jnp.int32)
    arg2 = jnp.zeros((BLOCK_B, D), jnp.int32)

    def body(kk, carry):
        m1, a1, m2, a2 = carry
        k = K - 1 - kk
        crow = c_ref[pl.ds(k, 1), :]
        d = x - crow
        d = d * d
        le1 = d <= m1
        le2 = d <= m2
        ki = jnp.full((), k, jnp.int32)
        new_m2 = jnp.minimum(m2, jnp.maximum(d, m1))
        new_a2 = jnp.where(le1, a1, jnp.where(le2, ki, a2))
        new_m1 = jnp.minimum(m1, d)
        new_a1 = jnp.where(le1, ki, a1)
        return new_m1, new_a1, new_m2, new_a2

    arg1, arg2 = lax.fori_loop(0, K, body, (min1, arg1, min2, arg2),
                               unroll=16)[1::2]

    out1_ref[...] = jnp.transpose(
        arg1.reshape(BLOCK_B // 16, 16, D), (0, 2, 1))
    out2_ref[...] = jnp.transpose(
        arg2.reshape(BLOCK_B // 16, 16, D), (0, 2, 1))


def _sc_mode_kernel(a1_hbm, a2_hbm, o1_hbm, o2_hbm, rows_v, counts_v,
                    mode_v):
    wid = lax.axis_index("s") * 2 + lax.axis_index("c")
    second = wid >= 16
    w = wid % 16

    lanes = lax.iota(jnp.int32, 16)
    zero16 = jnp.zeros((16,), jnp.int32)

    def zero_body(b, _):
        counts_v[pl.ds(b * 16, 16)] = zero16
        return 0

    lax.fori_loop(0, K, zero_body, 0)

    def group_body(g, _):
        grp = w * GPW + g

        @pl.when(second)
        def _():
            pltpu.sync_copy(a2_hbm.at[grp], rows_v)

        @pl.when(jnp.logical_not(second))
        def _():
            pltpu.sync_copy(a1_hbm.at[grp], rows_v)

        epoch = g * 256

        def d_body(d, best):
            idx = rows_v[d]
            flat = idx * 16 + lanes
            cur = plsc.load_gather(counts_v, [flat])
            cnt = jnp.maximum(cur - epoch, 0) + 1
            plsc.store_scatter(counts_v, [flat], cnt + epoch)
            key = cnt * 256 + (255 - idx)
            return jnp.maximum(best, key)

        best = lax.fori_loop(0, D, d_body, zero16)
        mode_v[...] = 255 - (best & 255)

        @pl.when(second)
        def _():
            pltpu.sync_copy(mode_v, o2_hbm.at[pl.ds(grp * 16, 16)])

        @pl.when(jnp.logical_not(second))
        def _():
            pltpu.sync_copy(mode_v, o1_hbm.at[pl.ds(grp * 16, 16)])

        return 0

    lax.fori_loop(0, GPW, group_body, 0)


LOSS_B = 256


def _loss_kernel(x_ref, c_ref, pos_ref, neg_ref, out_ref):
    x = x_ref[...]
    i = pl.program_id(0)

    iota_k = lax.broadcasted_iota(jnp.int32, (LOSS_B, K), 1)
    oh_p = (iota_k == pos_ref[...]).astype(jnp.float32)
    oh_n = (iota_k == neg_ref[...]).astype(jnp.float32)
    pos = lax.dot(oh_p, c_ref[...], preferred_element_type=jnp.float32)
    neg = lax.dot(oh_n, c_ref[...], preferred_element_type=jnp.float32)

    eps = jnp.float32(1e-6)
    dap = x - pos + eps
    dan = x - neg + eps
    dpn = pos - neg + eps
    d_ap = jnp.sqrt(jnp.sum(dap * dap, axis=1, keepdims=True))
    d_an = jnp.sqrt(jnp.sum(dan * dan, axis=1, keepdims=True))
    d_pn = jnp.sqrt(jnp.sum(dpn * dpn, axis=1, keepdims=True))
    d_neg = jnp.minimum(d_an, d_pn)
    partial = jnp.sum(jnp.maximum(d_ap - d_neg + 1.0, 0.0),
                      axis=0, keepdims=True)

    @pl.when(i == 0)
    def _():
        out_ref[...] = jnp.zeros((1, 1), jnp.float32)

    out_ref[...] += partial


def _sc_mode(argsT1, argsT2):
    fn = functools.partial(
        pl.kernel,
        out_type=[jax.ShapeDtypeStruct((BH,), jnp.int32),
                  jax.ShapeDtypeStruct((BH,), jnp.int32)],
        scratch_types=[pltpu.VMEM((D, 16), jnp.int32),
                       pltpu.VMEM((K * 16,), jnp.int32),
                       pltpu.VMEM((16,), jnp.int32)],
        mesh=plsc.VectorSubcoreMesh(core_axis_name="c",
                                    subcore_axis_name="s"),
        compiler_params=pltpu.CompilerParams(needs_layout_passes=False),
    )(_sc_mode_kernel)
    return fn(argsT1, argsT2)


def _scan(xh, centroids):
    return pl.pallas_call(
        _scan_kernel,
        grid=(BH // BLOCK_B,),
        in_specs=[
            pl.BlockSpec((BLOCK_B, D), lambda i: (i, 0)),
            pl.BlockSpec((K, D), lambda i: (0, 0)),
        ],
        out_specs=[
            pl.BlockSpec((BLOCK_B // 16, D, 16), lambda i: (i, 0, 0)),
            pl.BlockSpec((BLOCK_B // 16, D, 16), lambda i: (i, 0, 0)),
        ],
        out_shape=[
            jax.ShapeDtypeStruct((GROUPS, D, 16), jnp.int32),
            jax.ShapeDtypeStruct((GROUPS, D, 16), jnp.int32),
        ],
    )(xh, centroids)


def _loss(xh, centroids, pos_idx, neg_idx):
    return pl.pallas_call(
        _loss_kernel,
        grid=(BH // LOSS_B,),
        in_specs=[
            pl.BlockSpec((LOSS_B, D), lambda i: (i, 0)),
            pl.BlockSpec((K, D), lambda i: (0, 0)),
            pl.BlockSpec((LOSS_B, 1), lambda i: (i, 0)),
            pl.BlockSpec((LOSS_B, 1), lambda i: (i, 0)),
        ],
        out_specs=pl.BlockSpec((1, 1), lambda i: (0, 0)),
        out_shape=jax.ShapeDtypeStruct((1, 1), jnp.float32),
    )(xh, centroids, pos_idx.reshape(BH, 1), neg_idx.reshape(BH, 1))


@jax.jit
def kernel(input_features, centroids):
    halves = [input_features[h * BH:(h + 1) * BH] for h in range(HALVES)]
    args = [_scan(xh, centroids) for xh in halves]
    modes = [_sc_mode(a1, a2) for (a1, a2) in args]
    parts = [_loss(xh, centroids, p, n)
             for xh, (p, n) in zip(halves, modes)]
    total = sum(p[0, 0] for p in parts)
    return total / B


# speedup vs baseline: 1.0034x; 1.0034x over previous
import functools

import jax
import jax.numpy as jnp
from jax import lax
from jax.experimental import pallas as pl
from jax.experimental.pallas import tpu as pltpu
from jax.experimental.pallas import tpu_sc as plsc

B, K, D = 2048, 256, 128
BLOCK_B = 64
HALVES = 2
BH = B // HALVES
GROUPS = BH // 16
NW = 32
GPW = GROUPS // (NW // 2)


def _scan_kernel(x_ref, c_ref, out1_ref, out2_ref):
    x = x_ref[...]

    inf = jnp.float32(jnp.inf)
    min1 = jnp.full((BLOCK_B, D), inf, jnp.float32)
    min2 = jnp.full((BLOCK_B, D), inf, jnp.float32)
    arg1 = jnp.zeros((BLOCK_B, D), jnp.int32)
    arg2 = jnp.zeros((BLOCK_B, D), jnp.int32)

    def body(kk, carry):
        m1, a1, m2, a2 = carry
        k = K - 1 - kk
        crow = c_ref[pl.ds(k, 1), :]
        d = x - crow
        d = d * d
        le1 = d <= m1
        le2 = d <= m2
        ki = jnp.full((), k, jnp.int32)
        new_m2 = jnp.minimum(m2, jnp.maximum(d, m1))
        new_a2 = jnp.where(le1, a1, jnp.where(le2, ki, a2))
        new_m1 = jnp.minimum(m1, d)
        new_a1 = jnp.where(le1, ki, a1)
        return new_m1, new_a1, new_m2, new_a2

    arg1, arg2 = lax.fori_loop(0, K, body, (min1, arg1, min2, arg2),
                               unroll=16)[1::2]

    out1_ref[...] = jnp.transpose(
        arg1.reshape(BLOCK_B // 16, 16, D), (0, 2, 1))
    out2_ref[...] = jnp.transpose(
        arg2.reshape(BLOCK_B // 16, 16, D), (0, 2, 1))


def _sc_mode_kernel(a1_hbm, a2_hbm, o1_hbm, o2_hbm, rows_v, counts_v,
                    mode_v):
    wid = lax.axis_index("s") * 2 + lax.axis_index("c")
    second = wid >= 16
    w = wid % 16

    lanes = lax.iota(jnp.int32, 16)
    zero16 = jnp.zeros((16,), jnp.int32)

    def zero_body(b, _):
        counts_v[pl.ds(b * 16, 16)] = zero16
        return 0

    lax.fori_loop(0, K, zero_body, 0)

    def group_body(g, _):
        grp = w * GPW + g

        @pl.when(second)
        def _():
            pltpu.sync_copy(a2_hbm.at[grp], rows_v)

        @pl.when(jnp.logical_not(second))
        def _():
            pltpu.sync_copy(a1_hbm.at[grp], rows_v)

        epoch = g * 256

        def d_body(d, best):
            idx = rows_v[d]
            flat = idx * 16 + lanes
            cur = plsc.load_gather(counts_v, [flat])
            cnt = jnp.maximum(cur - epoch, 0) + 1
            plsc.store_scatter(counts_v, [flat], cnt + epoch)
            key = cnt * 256 + (255 - idx)
            return jnp.maximum(best, key)

        best = lax.fori_loop(0, D, d_body, zero16)
        mode_v[...] = 255 - (best & 255)

        @pl.when(second)
        def _():
            pltpu.sync_copy(mode_v, o2_hbm.at[pl.ds(grp * 16, 16)])

        @pl.when(jnp.logical_not(second))
        def _():
            pltpu.sync_copy(mode_v, o1_hbm.at[pl.ds(grp * 16, 16)])

        return 0

    lax.fori_loop(0, GPW, group_body, 0)


LOSS_B = 256


def _loss_kernel(x_ref, c_ref, pos_ref, neg_ref, out_ref):
    x = x_ref[...]
    i = pl.program_id(0)

    iota_k = lax.broadcasted_iota(jnp.int32, (LOSS_B, K), 1)
    oh_p = (iota_k == pos_ref[...]).astype(jnp.float32)
    oh_n = (iota_k == neg_ref[...]).astype(jnp.float32)
    pos = lax.dot(oh_p, c_ref[...], preferred_element_type=jnp.float32)
    neg = lax.dot(oh_n, c_ref[...], preferred_element_type=jnp.float32)

    eps = jnp.float32(1e-6)
    dap = x - pos + eps
    dan = x - neg + eps
    dpn = pos - neg + eps
    d_ap = jnp.sqrt(jnp.sum(dap * dap, axis=1, keepdims=True))
    d_an = jnp.sqrt(jnp.sum(dan * dan, axis=1, keepdims=True))
    d_pn = jnp.sqrt(jnp.sum(dpn * dpn, axis=1, keepdims=True))
    d_neg = jnp.minimum(d_an, d_pn)
    partial = jnp.sum(jnp.maximum(d_ap - d_neg + 1.0, 0.0),
                      axis=0, keepdims=True)

    @pl.when(i == 0)
    def _():
        out_ref[...] = jnp.zeros((1, 1), jnp.float32)

    out_ref[...] += partial


def _sc_mode(argsT1, argsT2):
    fn = functools.partial(
        pl.kernel,
        out_type=[jax.ShapeDtypeStruct((BH,), jnp.int32),
                  jax.ShapeDtypeStruct((BH,), jnp.int32)],
        scratch_types=[pltpu.VMEM((D, 16), jnp.int32),
                       pltpu.VMEM((K * 16,), jnp.int32),
                       pltpu.VMEM((16,), jnp.int32)],
        mesh=plsc.VectorSubcoreMesh(core_axis_name="c",
                                    subcore_axis_name="s"),
        compiler_params=pltpu.CompilerParams(needs_layout_passes=False),
    )(_sc_mode_kernel)
    return fn(argsT1, argsT2)


def _scan(xh, centroids):
    return pl.pallas_call(
        _scan_kernel,
        grid=(BH // BLOCK_B,),
        in_specs=[
            pl.BlockSpec((BLOCK_B, D), lambda i: (i, 0)),
            pl.BlockSpec((K, D), lambda i: (0, 0)),
        ],
        out_specs=[
            pl.BlockSpec((BLOCK_B // 16, D, 16), lambda i: (i, 0, 0)),
            pl.BlockSpec((BLOCK_B // 16, D, 16), lambda i: (i, 0, 0)),
        ],
        out_shape=[
            jax.ShapeDtypeStruct((GROUPS, D, 16), jnp.int32),
            jax.ShapeDtypeStruct((GROUPS, D, 16), jnp.int32),
        ],
    )(xh, centroids)


def _loss(xh, centroids, pos_idx, neg_idx):
    return pl.pallas_call(
        _loss_kernel,
        grid=(BH // LOSS_B,),
        in_specs=[
            pl.BlockSpec((LOSS_B, D), lambda i: (i, 0)),
            pl.BlockSpec((K, D), lambda i: (0, 0)),
            pl.BlockSpec((LOSS_B, 1), lambda i: (i, 0)),
            pl.BlockSpec((LOSS_B, 1), lambda i: (i, 0)),
        ],
        out_specs=pl.BlockSpec((1, 1), lambda i: (0, 0)),
        out_shape=jax.ShapeDtypeStruct((1, 1), jnp.float32),
    )(xh, centroids, pos_idx.reshape(BH, 1), neg_idx.reshape(BH, 1))


@jax.jit
def kernel(input_features, centroids):
    halves = [input_features[h * BH:(h + 1) * BH] for h in range(HALVES)]
    args = [_scan(xh, centroids) for xh in halves]
    modes = [_sc_mode(a1, a2) for (a1, a2) in args]
    parts = [_loss(xh, centroids, p, n)
             for xh, (p, n) in zip(halves, modes)]
    total = sum(p[0, 0] for p in parts)
    return total / B


# speedup vs baseline: 1.0110x; 1.0076x over previous
import functools

import jax
import jax.numpy as jnp
from jax import lax
from jax.experimental import pallas as pl
from jax.experimental.pallas import tpu as pltpu
from jax.experimental.pallas import tpu_sc as plsc

B, K, D = 2048, 256, 128
BLOCK_B = 64
HALVES = 2
BH = B // HALVES
GROUPS = BH // 16
NW = 32
GPW = GROUPS // (NW // 2)


def _scan_kernel(x_ref, c_ref, out1_ref, out2_ref):
    x = x_ref[...]

    inf = jnp.float32(jnp.inf)
    min1 = jnp.full((BLOCK_B, D), inf, jnp.float32)
    min2 = jnp.full((BLOCK_B, D), inf, jnp.float32)
    arg1 = jnp.zeros((BLOCK_B, D), jnp.int32)
    arg2 = jnp.zeros((BLOCK_B, D), jnp.int32)

    def body(kk, carry):
        m1, a1, m2, a2 = carry
        k = K - 1 - kk
        crow = c_ref[pl.ds(k, 1), :]
        d = x - crow
        d = d * d
        le1 = d <= m1
        le2 = d <= m2
        ki = jnp.full((), k, jnp.int32)
        new_m2 = jnp.minimum(m2, jnp.maximum(d, m1))
        new_a2 = jnp.where(le1, a1, jnp.where(le2, ki, a2))
        new_m1 = jnp.minimum(m1, d)
        new_a1 = jnp.where(le1, ki, a1)
        return new_m1, new_a1, new_m2, new_a2

    arg1, arg2 = lax.fori_loop(0, K, body, (min1, arg1, min2, arg2),
                               unroll=32)[1::2]

    out1_ref[...] = jnp.transpose(
        arg1.reshape(BLOCK_B // 16, 16, D), (0, 2, 1))
    out2_ref[...] = jnp.transpose(
        arg2.reshape(BLOCK_B // 16, 16, D), (0, 2, 1))


def _sc_mode_kernel(a1_hbm, a2_hbm, o1_hbm, o2_hbm, rows_v, counts_v,
                    mode_v):
    wid = lax.axis_index("s") * 2 + lax.axis_index("c")
    second = wid >= 16
    w = wid % 16

    lanes = lax.iota(jnp.int32, 16)
    zero16 = jnp.zeros((16,), jnp.int32)

    def zero_body(b, _):
        counts_v[pl.ds(b * 16, 16)] = zero16
        return 0

    lax.fori_loop(0, K, zero_body, 0)

    def group_body(g, _):
        grp = w * GPW + g

        @pl.when(second)
        def _():
            pltpu.sync_copy(a2_hbm.at[grp], rows_v)

        @pl.when(jnp.logical_not(second))
        def _():
            pltpu.sync_copy(a1_hbm.at[grp], rows_v)

        epoch = g * 256

        def d_body(d, best):
            idx = rows_v[d]
            flat = idx * 16 + lanes
            cur = plsc.load_gather(counts_v, [flat])
            cnt = jnp.maximum(cur - epoch, 0) + 1
            plsc.store_scatter(counts_v, [flat], cnt + epoch)
            key = cnt * 256 + (255 - idx)
            return jnp.maximum(best, key)

        best = lax.fori_loop(0, D, d_body, zero16)
        mode_v[...] = 255 - (best & 255)

        @pl.when(second)
        def _():
            pltpu.sync_copy(mode_v, o2_hbm.at[pl.ds(grp * 16, 16)])

        @pl.when(jnp.logical_not(second))
        def _():
            pltpu.sync_copy(mode_v, o1_hbm.at[pl.ds(grp * 16, 16)])

        return 0

    lax.fori_loop(0, GPW, group_body, 0)


LOSS_B = 256


def _loss_kernel(x_ref, c_ref, pos_ref, neg_ref, out_ref):
    x = x_ref[...]
    i = pl.program_id(0)

    iota_k = lax.broadcasted_iota(jnp.int32, (LOSS_B, K), 1)
    oh_p = (iota_k == pos_ref[...]).astype(jnp.float32)
    oh_n = (iota_k == neg_ref[...]).astype(jnp.float32)
    pos = lax.dot(oh_p, c_ref[...], preferred_element_type=jnp.float32)
    neg = lax.dot(oh_n, c_ref[...], preferred_element_type=jnp.float32)

    eps = jnp.float32(1e-6)
    dap = x - pos + eps
    dan = x - neg + eps
    dpn = pos - neg + eps
    d_ap = jnp.sqrt(jnp.sum(dap * dap, axis=1, keepdims=True))
    d_an = jnp.sqrt(jnp.sum(dan * dan, axis=1, keepdims=True))
    d_pn = jnp.sqrt(jnp.sum(dpn * dpn, axis=1, keepdims=True))
    d_neg = jnp.minimum(d_an, d_pn)
    partial = jnp.sum(jnp.maximum(d_ap - d_neg + 1.0, 0.0),
                      axis=0, keepdims=True)

    @pl.when(i == 0)
    def _():
        out_ref[...] = jnp.zeros((1, 1), jnp.float32)

    out_ref[...] += partial


def _sc_mode(argsT1, argsT2):
    fn = functools.partial(
        pl.kernel,
        out_type=[jax.ShapeDtypeStruct((BH,), jnp.int32),
                  jax.ShapeDtypeStruct((BH,), jnp.int32)],
        scratch_types=[pltpu.VMEM((D, 16), jnp.int32),
                       pltpu.VMEM((K * 16,), jnp.int32),
                       pltpu.VMEM((16,), jnp.int32)],
        mesh=plsc.VectorSubcoreMesh(core_axis_name="c",
                                    subcore_axis_name="s"),
        compiler_params=pltpu.CompilerParams(needs_layout_passes=False),
    )(_sc_mode_kernel)
    return fn(argsT1, argsT2)


def _scan(xh, centroids):
    return pl.pallas_call(
        _scan_kernel,
        grid=(BH // BLOCK_B,),
        in_specs=[
            pl.BlockSpec((BLOCK_B, D), lambda i: (i, 0)),
            pl.BlockSpec((K, D), lambda i: (0, 0)),
        ],
        out_specs=[
            pl.BlockSpec((BLOCK_B // 16, D, 16), lambda i: (i, 0, 0)),
            pl.BlockSpec((BLOCK_B // 16, D, 16), lambda i: (i, 0, 0)),
        ],
        out_shape=[
            jax.ShapeDtypeStruct((GROUPS, D, 16), jnp.int32),
            jax.ShapeDtypeStruct((GROUPS, D, 16), jnp.int32),
        ],
    )(xh, centroids)


def _loss(xh, centroids, pos_idx, neg_idx):
    return pl.pallas_call(
        _loss_kernel,
        grid=(BH // LOSS_B,),
        in_specs=[
            pl.BlockSpec((LOSS_B, D), lambda i: (i, 0)),
            pl.BlockSpec((K, D), lambda i: (0, 0)),
            pl.BlockSpec((LOSS_B, 1), lambda i: (i, 0)),
            pl.BlockSpec((LOSS_B, 1), lambda i: (i, 0)),
        ],
        out_specs=pl.BlockSpec((1, 1), lambda i: (0, 0)),
        out_shape=jax.ShapeDtypeStruct((1, 1), jnp.float32),
    )(xh, centroids, pos_idx.reshape(BH, 1), neg_idx.reshape(BH, 1))


@jax.jit
def kernel(input_features, centroids):
    halves = [input_features[h * BH:(h + 1) * BH] for h in range(HALVES)]
    args = [_scan(xh, centroids) for xh in halves]
    modes = [_sc_mode(a1, a2) for (a1, a2) in args]
    parts = [_loss(xh, centroids, p, n)
             for xh, (p, n) in zip(halves, modes)]
    total = sum(p[0, 0] for p in parts)
    return total / B


# speedup vs baseline: 1.0239x; 1.0128x over previous
import functools

import jax
import jax.numpy as jnp
from jax import lax
from jax.experimental import pallas as pl
from jax.experimental.pallas import tpu as pltpu
from jax.experimental.pallas import tpu_sc as plsc

B, K, D = 2048, 256, 128
BLOCK_B = 64
HALVES = 2
BH = B // HALVES
GROUPS = BH // 16
NW = 32
GPW = GROUPS // (NW // 2)


def _scan_kernel(x_ref, c_ref, out1_ref, out2_ref):
    x = x_ref[...]

    inf = jnp.float32(jnp.inf)
    min1 = jnp.full((BLOCK_B, D), inf, jnp.float32)
    min2 = jnp.full((BLOCK_B, D), inf, jnp.float32)
    arg1 = jnp.zeros((BLOCK_B, D), jnp.int32)
    arg2 = jnp.zeros((BLOCK_B, D), jnp.int32)

    def body(kk, carry):
        m1, a1, m2, a2 = carry
        k = K - 1 - kk
        crow = c_ref[pl.ds(k, 1), :]
        d = x - crow
        d = d * d
        le1 = d <= m1
        le2 = d <= m2
        ki = jnp.full((), k, jnp.int32)
        new_m2 = jnp.minimum(m2, jnp.maximum(d, m1))
        new_a2 = jnp.where(le1, a1, jnp.where(le2, ki, a2))
        new_m1 = jnp.minimum(m1, d)
        new_a1 = jnp.where(le1, ki, a1)
        return new_m1, new_a1, new_m2, new_a2

    arg1, arg2 = lax.fori_loop(0, K, body, (min1, arg1, min2, arg2),
                               unroll=32)[1::2]

    out1_ref[...] = jnp.transpose(
        arg1.reshape(BLOCK_B // 16, 16, D), (0, 2, 1))
    out2_ref[...] = jnp.transpose(
        arg2.reshape(BLOCK_B // 16, 16, D), (0, 2, 1))


def _sc_mode_kernel(a1_hbm, a2_hbm, o1_hbm, o2_hbm, rows_v, counts_v,
                    mode_v):
    wid = lax.axis_index("s") * 2 + lax.axis_index("c")
    second = wid >= 16
    w = wid % 16

    lanes = lax.iota(jnp.int32, 16)
    zero16 = jnp.zeros((16,), jnp.int32)

    def zero_body(b, _):
        counts_v[pl.ds(b * 16, 16)] = zero16
        return 0

    lax.fori_loop(0, K, zero_body, 0)

    def group_body(g, _):
        grp = w * GPW + g

        @pl.when(second)
        def _():
            pltpu.sync_copy(a2_hbm.at[grp], rows_v)

        @pl.when(jnp.logical_not(second))
        def _():
            pltpu.sync_copy(a1_hbm.at[grp], rows_v)

        epoch = g * 256

        def d_body(d, best):
            idx = rows_v[d]
            flat = idx * 16 + lanes
            cur = plsc.load_gather(counts_v, [flat])
            cnt = jnp.maximum(cur - epoch, 0) + 1
            plsc.store_scatter(counts_v, [flat], cnt + epoch)
            key = cnt * 256 + (255 - idx)
            return jnp.maximum(best, key)

        best = lax.fori_loop(0, D, d_body, zero16)
        mode_v[...] = 255 - (best & 255)

        @pl.when(second)
        def _():
            pltpu.sync_copy(mode_v, o2_hbm.at[pl.ds(grp * 16, 16)])

        @pl.when(jnp.logical_not(second))
        def _():
            pltpu.sync_copy(mode_v, o1_hbm.at[pl.ds(grp * 16, 16)])

        return 0

    lax.fori_loop(0, GPW, group_body, 0)


LOSS_B = 256


def _loss_kernel(x_ref, c_ref, pos_ref, neg_ref, out_ref):
    x = x_ref[...]
    i = pl.program_id(0)

    iota_k = lax.broadcasted_iota(jnp.int32, (LOSS_B, K), 1)
    oh_p = (iota_k == pos_ref[...]).astype(jnp.float32)
    oh_n = (iota_k == neg_ref[...]).astype(jnp.float32)
    pos = lax.dot(oh_p, c_ref[...], preferred_element_type=jnp.float32)
    neg = lax.dot(oh_n, c_ref[...], preferred_element_type=jnp.float32)

    eps = jnp.float32(1e-6)
    dap = x - pos + eps
    dan = x - neg + eps
    dpn = pos - neg + eps
    d_ap = jnp.sqrt(jnp.sum(dap * dap, axis=1, keepdims=True))
    d_an = jnp.sqrt(jnp.sum(dan * dan, axis=1, keepdims=True))
    d_pn = jnp.sqrt(jnp.sum(dpn * dpn, axis=1, keepdims=True))
    d_neg = jnp.minimum(d_an, d_pn)
    partial = jnp.sum(jnp.maximum(d_ap - d_neg + 1.0, 0.0),
                      axis=0, keepdims=True)

    @pl.when(i == 0)
    def _():
        out_ref[...] = jnp.zeros((1, 1), jnp.float32)

    out_ref[...] += partial


def _sc_mode(argsT1, argsT2):
    fn = functools.partial(
        pl.kernel,
        out_type=[jax.ShapeDtypeStruct((BH,), jnp.int32),
                  jax.ShapeDtypeStruct((BH,), jnp.int32)],
        scratch_types=[pltpu.VMEM((D, 16), jnp.int32),
                       pltpu.VMEM((K * 16,), jnp.int32),
                       pltpu.VMEM((16,), jnp.int32)],
        mesh=plsc.VectorSubcoreMesh(core_axis_name="c",
                                    subcore_axis_name="s"),
        compiler_params=pltpu.CompilerParams(needs_layout_passes=False),
    )(_sc_mode_kernel)
    return fn(argsT1, argsT2)


def _scan(xh, centroids):
    return pl.pallas_call(
        _scan_kernel,
        grid=(BH // BLOCK_B,),
        in_specs=[
            pl.BlockSpec((BLOCK_B, D), lambda i: (i, 0)),
            pl.BlockSpec((K, D), lambda i: (0, 0)),
        ],
        out_specs=[
            pl.BlockSpec((BLOCK_B // 16, D, 16), lambda i: (i, 0, 0)),
            pl.BlockSpec((BLOCK_B // 16, D, 16), lambda i: (i, 0, 0)),
        ],
        out_shape=[
            jax.ShapeDtypeStruct((GROUPS, D, 16), jnp.int32),
            jax.ShapeDtypeStruct((GROUPS, D, 16), jnp.int32),
        ],
    )(xh, centroids)


def _loss(x, centroids, pos_idx, neg_idx):
    return pl.pallas_call(
        _loss_kernel,
        grid=(B // LOSS_B,),
        in_specs=[
            pl.BlockSpec((LOSS_B, D), lambda i: (i, 0)),
            pl.BlockSpec((K, D), lambda i: (0, 0)),
            pl.BlockSpec((LOSS_B, 1), lambda i: (i, 0)),
            pl.BlockSpec((LOSS_B, 1), lambda i: (i, 0)),
        ],
        out_specs=pl.BlockSpec((1, 1), lambda i: (0, 0)),
        out_shape=jax.ShapeDtypeStruct((1, 1), jnp.float32),
    )(x, centroids, pos_idx.reshape(B, 1), neg_idx.reshape(B, 1))


@jax.jit
def kernel(input_features, centroids):
    halves = [input_features[h * BH:(h + 1) * BH] for h in range(HALVES)]
    args = [_scan(xh, centroids) for xh in halves]
    modes = [_sc_mode(a1, a2) for (a1, a2) in args]
    pos_idx = jnp.concatenate([m[0] for m in modes])
    neg_idx = jnp.concatenate([m[1] for m in modes])
    total = _loss(input_features, centroids, pos_idx, neg_idx)
    return total[0, 0] / B
